# Initial kernel scaffold; baseline (speedup 1.0000x reference)
import jax
import jax.numpy as jnp
from jax.experimental import pallas as pl


def kernel(x, W, u, c, bias, adj):
    raise NotImplementedError("write your pallas kernel here")


# speedup vs baseline: 7.8609x; 7.8609x over previous
import functools

import jax
import jax.numpy as jnp
from jax import lax
from jax.experimental import pallas as pl
from jax.experimental.pallas import tpu as pltpu
from jax.experimental.pallas import tpu_sc as plsc

N_NODES = 10000
IN_C = 128
OUT_C = 64

NC = 2
NS = 16
NW = NC * NS
L = 16

K = 128
ROWS_PER_TILE = 632
SPAD = ROWS_PER_TILE * NS
TRASH = N_NODES
NTRASH = SPAD - N_NODES


def _mm_body(x_ref, w_ref, o_ref):
    o_ref[...] = jnp.dot(x_ref[...], w_ref[...],
                         preferred_element_type=jnp.float32)


def _matmul(x, W):
    blk = 1000
    grid = N_NODES // blk
    return pl.pallas_call(
        _mm_body,
        grid=(grid,),
        in_specs=[
            pl.BlockSpec((blk, IN_C), lambda i: (i, 0)),
            pl.BlockSpec((IN_C, OUT_C), lambda i: (0, 0)),
        ],
        out_specs=pl.BlockSpec((blk, OUT_C), lambda i: (i, 0)),
        out_shape=jax.ShapeDtypeStruct((N_NODES, OUT_C), jnp.float32),
    )(x, W)


def _fin_body(p_ref, c_ref, h_ref, b_ref, o_ref):
    s = p_ref[0] + p_ref[1] + h_ref[...]
    cnt = c_ref[0] + c_ref[1] + 1.0
    o_ref[...] = jnp.maximum(s / cnt + b_ref[...], 0.0)


def _finalize(psum, pcnt, h, bias2d):
    blk = 1000
    grid = N_NODES // blk
    return pl.pallas_call(
        _fin_body,
        grid=(grid,),
        in_specs=[
            pl.BlockSpec((NC, blk, OUT_C), lambda i: (0, i, 0)),
            pl.BlockSpec((NC, blk, 1), lambda i: (0, i, 0)),
            pl.BlockSpec((blk, OUT_C), lambda i: (i, 0)),
            pl.BlockSpec((1, OUT_C), lambda i: (0, 0)),
        ],
        out_specs=pl.BlockSpec((blk, OUT_C), lambda i: (i, 0)),
        out_shape=jax.ShapeDtypeStruct((N_NODES, OUT_C), jnp.float32),
    )(psum, pcnt, h, bias2d)


def _make_sc_aggregate(e_pad):
    chunks_per_worker = e_pad // (NW * K)
    mesh = plsc.VectorSubcoreMesh(core_axis_name="c", subcore_axis_name="s")

    @functools.partial(
        pl.kernel,
        out_type=[
            jax.ShapeDtypeStruct((NC * SPAD, OUT_C), jnp.float32),
            jax.ShapeDtypeStruct((NC * SPAD,), jnp.float32),
        ],
        mesh=mesh,
        compiler_params=pltpu.CompilerParams(use_tc_tiling_on_sc=False),
        scratch_types=[
            pltpu.VMEM((K,), jnp.int32),
            pltpu.VMEM((K,), jnp.int32),
            pltpu.VMEM((K,), jnp.int32),
            pltpu.VMEM((K, OUT_C), jnp.float32),
            pltpu.VMEM((K,), jnp.float32),
            pltpu.VMEM((ROWS_PER_TILE, OUT_C), jnp.float32),
            pltpu.VMEM((640,), jnp.float32),
            pltpu.VMEM_SHARED((SPAD, OUT_C), jnp.float32),
            pltpu.VMEM_SHARED((SPAD,), jnp.float32),
            pltpu.SemaphoreType.DMA,
        ],
    )
    def sc_aggregate(src_hbm, dst_hbm, h_hbm, psum_hbm, pcnt_hbm,
                     src_v, dst_v, dstp_v, rows_v, ones_v, zblk_v, zcnt_v,
                     ssum, scnt, gsem):
        cid = lax.axis_index("c")
        sid = lax.axis_index("s")
        wid = sid * NC + cid

        zero16 = jnp.zeros((L,), jnp.float32)
        one16 = jnp.full((L,), 1.0, jnp.float32)

        def fill_ones(i, _):
            ones_v[pl.ds(i * L, L)] = one16
            return 0
        lax.fori_loop(0, K // L, fill_ones, 0)

        sub = OUT_C // L

        def zb(t, _):
            zblk_v[t // sub, pl.ds((t % sub) * L, L)] = zero16
            return 0
        lax.fori_loop(0, ROWS_PER_TILE * sub, zb, 0)

        def zc(i, _):
            zcnt_v[pl.ds(i * L, L)] = zero16
            return 0
        lax.fori_loop(0, 640 // L, zc, 0)

        rbase = sid * ROWS_PER_TILE
        pltpu.sync_copy(zblk_v, ssum.at[pl.ds(rbase, ROWS_PER_TILE)])
        pltpu.sync_copy(zcnt_v.at[pl.ds(0, ROWS_PER_TILE)],
                        scnt.at[pl.ds(rbase, ROWS_PER_TILE)])
        plsc.subcore_barrier()

        ebase = wid * (chunks_per_worker * K)

        def step(ch, _):
            off = ebase + ch * K
            pltpu.sync_copy(src_hbm.at[pl.ds(off, K)], src_v)
            pltpu.sync_copy(dst_hbm.at[pl.ds(off, K)], dst_v)

            def mk(i, _):
                sv = src_v[pl.ds(i * L, L)]
                dv = dst_v[pl.ds(i * L, L)]
                trash = (TRASH + ((ch * (K // L) + i) % (NTRASH // L)) * L
                         + lax.iota(jnp.int32, L))
                dstp_v[pl.ds(i * L, L)] = jnp.where(sv == dv, trash, dv)
                return 0
            lax.fori_loop(0, K // L, mk, 0)

            pltpu.async_copy(h_hbm.at[src_v], rows_v, gsem).wait()
            pltpu.sync_copy(rows_v, ssum.at[dstp_v], add=True)
            pltpu.sync_copy(ones_v, scnt.at[dstp_v], add=True)
            return 0
        lax.fori_loop(0, chunks_per_worker, step, 0)

        plsc.subcore_barrier()
        obase = cid * SPAD + rbase
        pltpu.sync_copy(ssum.at[pl.ds(rbase, ROWS_PER_TILE)],
                        psum_hbm.at[pl.ds(obase, ROWS_PER_TILE)])
        pltpu.sync_copy(scnt.at[pl.ds(rbase, ROWS_PER_TILE)],
                        pcnt_hbm.at[pl.ds(obase, ROWS_PER_TILE)])

    return sc_aggregate


def kernel(x, W, u, c, bias, adj):
    del u, c
    h = _matmul(x, W)

    src = adj[0]
    dst = adj[1]
    n_edges = src.shape[0]
    e_pad = ((n_edges + NW * K - 1) // (NW * K)) * (NW * K)
    pad = e_pad - n_edges
    if pad:
        zpad = jnp.zeros((pad,), jnp.int32)
        src = jnp.concatenate([src, zpad])
        dst = jnp.concatenate([dst, zpad])

    psum, pcnt = _make_sc_aggregate(e_pad)(src, dst, h)

    return _finalize(psum.reshape(NC, SPAD, OUT_C), pcnt.reshape(NC, SPAD, 1),
                     h, bias.reshape(1, OUT_C))


# speedup vs baseline: 19.3152x; 2.4571x over previous
import functools

import jax
import jax.numpy as jnp
from jax import lax
from jax.experimental import pallas as pl
from jax.experimental.pallas import tpu as pltpu
from jax.experimental.pallas import tpu_sc as plsc

N_NODES = 10000
IN_C = 128
OUT_C = 64

NC = 2
NS = 16
NW = NC * NS
L = 16

K = 128
ROWS_PER_TILE = 632
SPAD = ROWS_PER_TILE * NS
TRASH = N_NODES
NTRASH = SPAD - N_NODES


def _mm_body(x_ref, w_ref, o_ref):
    o_ref[...] = jnp.dot(x_ref[...], w_ref[...],
                         preferred_element_type=jnp.float32)


def _matmul(x, W):
    blk = 1000
    grid = N_NODES // blk
    return pl.pallas_call(
        _mm_body,
        grid=(grid,),
        in_specs=[
            pl.BlockSpec((blk, IN_C), lambda i: (i, 0)),
            pl.BlockSpec((IN_C, OUT_C), lambda i: (0, 0)),
        ],
        out_specs=pl.BlockSpec((blk, OUT_C), lambda i: (i, 0)),
        out_shape=jax.ShapeDtypeStruct((N_NODES, OUT_C), jnp.float32),
    )(x, W)


def _fin_body(p_ref, c_ref, h_ref, b_ref, o_ref):
    s = p_ref[0] + p_ref[1] + h_ref[...]
    cnt = c_ref[0] + c_ref[1] + 1.0
    o_ref[...] = jnp.maximum(s / cnt + b_ref[...], 0.0)


def _finalize(psum, pcnt, h, bias2d):
    blk = 1000
    grid = N_NODES // blk
    return pl.pallas_call(
        _fin_body,
        grid=(grid,),
        in_specs=[
            pl.BlockSpec((NC, blk, OUT_C), lambda i: (0, i, 0)),
            pl.BlockSpec((NC, blk, 1), lambda i: (0, i, 0)),
            pl.BlockSpec((blk, OUT_C), lambda i: (i, 0)),
            pl.BlockSpec((1, OUT_C), lambda i: (0, 0)),
        ],
        out_specs=pl.BlockSpec((blk, OUT_C), lambda i: (i, 0)),
        out_shape=jax.ShapeDtypeStruct((N_NODES, OUT_C), jnp.float32),
    )(psum, pcnt, h, bias2d)


NBUF = 8
AHEAD = 4


def _make_sc_aggregate(cpw):
    assert cpw % NBUF == 0 and cpw >= 2 * NBUF
    n_groups = cpw // NBUF
    mesh = plsc.VectorSubcoreMesh(core_axis_name="c", subcore_axis_name="s")

    @functools.partial(
        pl.kernel,
        out_type=[
            jax.ShapeDtypeStruct((NC * SPAD, OUT_C), jnp.float32),
            jax.ShapeDtypeStruct((NC * SPAD,), jnp.float32),
        ],
        mesh=mesh,
        compiler_params=pltpu.CompilerParams(use_tc_tiling_on_sc=False),
        scratch_types=[
            pltpu.VMEM((cpw, K), jnp.int32),
            pltpu.VMEM((cpw, K), jnp.int32),
            [pltpu.VMEM((K, OUT_C), jnp.float32) for _ in range(NBUF)],
            pltpu.VMEM((K,), jnp.float32),
            pltpu.VMEM((640,), jnp.float32),
            pltpu.VMEM_SHARED((SPAD, OUT_C), jnp.float32),
            pltpu.VMEM_SHARED((SPAD,), jnp.float32),
            [pltpu.SemaphoreType.DMA for _ in range(NBUF)],
            [pltpu.SemaphoreType.DMA for _ in range(NBUF)],
            [pltpu.SemaphoreType.DMA for _ in range(NBUF)],
        ],
    )
    def sc_aggregate(src_hbm, dst_hbm, h_hbm, psum_hbm, pcnt_hbm,
                     src_all, dstp_all, rows, ones_v, zcnt_v,
                     ssum, scnt, gsems, ssems, csems):
        cid = lax.axis_index("c")
        sid = lax.axis_index("s")
        wid = sid * NC + cid

        zero16 = jnp.zeros((L,), jnp.float32)
        one16 = jnp.full((L,), 1.0, jnp.float32)

        pltpu.sync_copy(src_hbm.at[wid], src_all)
        pltpu.sync_copy(dst_hbm.at[wid], dstp_all)

        def fill_ones(i, _):
            ones_v[pl.ds(i * L, L)] = one16
            return 0
        lax.fori_loop(0, K // L, fill_ones, 0)

        lanes = lax.iota(jnp.int32, L)
        sub = K // L

        def mk_chunk(ch, _):
            def mk(i, _):
                sv = src_all[ch, pl.ds(i * L, L)]
                dv = dstp_all[ch, pl.ds(i * L, L)]
                trash = (TRASH + ((ch * sub + i) % (NTRASH // L)) * L
                         + lanes)
                dstp_all[ch, pl.ds(i * L, L)] = jnp.where(sv == dv, trash, dv)
                return 0
            return lax.fori_loop(0, sub, mk, 0)
        lax.fori_loop(0, cpw, mk_chunk, 0)

        zsub = OUT_C // L

        def zb(t, _):
            rows[0][t // zsub, pl.ds((t % zsub) * L, L)] = zero16
            return 0
        lax.fori_loop(0, K * zsub, zb, 0)

        def zc(i, _):
            zcnt_v[pl.ds(i * L, L)] = zero16
            return 0
        lax.fori_loop(0, 640 // L, zc, 0)

        rbase = sid * ROWS_PER_TILE
        nfull, rem = divmod(ROWS_PER_TILE, K)
        for t in range(nfull):
            pltpu.sync_copy(rows[0], ssum.at[pl.ds(rbase + t * K, K)])
        if rem:
            pltpu.sync_copy(rows[0].at[pl.ds(0, rem)],
                            ssum.at[pl.ds(rbase + nfull * K, rem)])
        pltpu.sync_copy(zcnt_v.at[pl.ds(0, ROWS_PER_TILE)],
                        scnt.at[pl.ds(rbase, ROWS_PER_TILE)])
        plsc.subcore_barrier()

        def issue_gather(ch, b):
            pltpu.async_copy(h_hbm.at[src_all.at[ch]], rows[b], gsems[b])

        def wait_gather(b):
            pltpu.make_async_copy(h_hbm.at[src_all.at[0]], rows[b],
                                  gsems[b]).wait()

        def issue_scatters(ch, b):
            pltpu.async_copy(rows[b], ssum.at[dstp_all.at[ch]], ssems[b],
                             add=True)
            pltpu.async_copy(ones_v, scnt.at[dstp_all.at[ch]], csems[b],
                             add=True)

        def wait_scatters(b):
            pltpu.make_async_copy(rows[b], ssum.at[dstp_all.at[0]],
                                  ssems[b]).wait()
            pltpu.make_async_copy(ones_v, scnt.at[dstp_all.at[0]],
                                  csems[b]).wait()

        for b in range(AHEAD):
            issue_gather(b, b)

        for b in range(NBUF):
            bg = (b + AHEAD) % NBUF
            if b >= AHEAD:
                wait_scatters(bg)
            issue_gather(b + AHEAD, bg)
            wait_gather(b)
            issue_scatters(b, b)

        def group(g, _):
            base = g * NBUF
            for b in range(NBUF):
                bg = (b + AHEAD) % NBUF
                wait_scatters(bg)
                issue_gather(base + b + AHEAD, bg)
                wait_gather(b)
                issue_scatters(base + b, b)
            return 0
        lax.fori_loop(1, n_groups - 1, group, 0)

        tbase = (n_groups - 1) * NBUF
        for b in range(NBUF):
            if b < AHEAD:
                bg = (b + AHEAD) % NBUF
                wait_scatters(bg)
                issue_gather(tbase + b + AHEAD, bg)
            wait_gather(b)
            issue_scatters(tbase + b, b)

        for b in range(NBUF):
            wait_scatters(b)

        plsc.subcore_barrier()
        obase = cid * SPAD + rbase
        pltpu.sync_copy(ssum.at[pl.ds(rbase, ROWS_PER_TILE)],
                        psum_hbm.at[pl.ds(obase, ROWS_PER_TILE)])
        pltpu.sync_copy(scnt.at[pl.ds(rbase, ROWS_PER_TILE)],
                        pcnt_hbm.at[pl.ds(obase, ROWS_PER_TILE)])

    return sc_aggregate


def kernel(x, W, u, c, bias, adj):
    del u, c
    h = _matmul(x, W)

    src = adj[0]
    dst = adj[1]
    n_edges = src.shape[0]
    quantum = NW * K * NBUF
    e_pad = max(((n_edges + quantum - 1) // quantum) * quantum, 2 * quantum)
    pad = e_pad - n_edges
    if pad:
        ppad = (jnp.arange(pad, dtype=jnp.int32) * 97) % N_NODES
        src = jnp.concatenate([src, ppad])
        dst = jnp.concatenate([dst, ppad])
    cpw = e_pad // (NW * K)

    psum, pcnt = _make_sc_aggregate(cpw)(
        src.reshape(NW, cpw, K), dst.reshape(NW, cpw, K), h)

    return _finalize(psum.reshape(NC, SPAD, OUT_C), pcnt.reshape(NC, SPAD, 1),
                     h, bias.reshape(1, OUT_C))


# speedup vs baseline: 19.4540x; 1.0072x over previous
import functools

import jax
import jax.numpy as jnp
from jax import lax
from jax.experimental import pallas as pl
from jax.experimental.pallas import tpu as pltpu
from jax.experimental.pallas import tpu_sc as plsc

N_NODES = 10000
IN_C = 128
OUT_C = 64

NC = 2
NS = 16
NW = NC * NS
L = 16

K = 128
ROWS_PER_TILE = 632
SPAD = ROWS_PER_TILE * NS
TRASH = N_NODES
NTRASH = SPAD - N_NODES


def _mm_body(x_ref, w_ref, o_ref):
    o_ref[...] = jnp.dot(x_ref[...], w_ref[...],
                         preferred_element_type=jnp.float32)


def _matmul(x, W):
    blk = 1000
    grid = N_NODES // blk
    return pl.pallas_call(
        _mm_body,
        grid=(grid,),
        in_specs=[
            pl.BlockSpec((blk, IN_C), lambda i: (i, 0)),
            pl.BlockSpec((IN_C, OUT_C), lambda i: (0, 0)),
        ],
        out_specs=pl.BlockSpec((blk, OUT_C), lambda i: (i, 0)),
        out_shape=jax.ShapeDtypeStruct((N_NODES, OUT_C), jnp.float32),
    )(x, W)


def _fin_body(p_ref, c_ref, h_ref, b_ref, o_ref):
    s = p_ref[0] + p_ref[1] + h_ref[...]
    cnt = c_ref[0] + c_ref[1] + 1.0
    o_ref[...] = jnp.maximum(s / cnt + b_ref[...], 0.0)


def _finalize(psum, pcnt, h, bias2d):
    blk = 1000
    grid = N_NODES // blk
    return pl.pallas_call(
        _fin_body,
        grid=(grid,),
        in_specs=[
            pl.BlockSpec((NC, blk, OUT_C), lambda i: (0, i, 0)),
            pl.BlockSpec((NC, blk, 1), lambda i: (0, i, 0)),
            pl.BlockSpec((blk, OUT_C), lambda i: (i, 0)),
            pl.BlockSpec((1, OUT_C), lambda i: (0, 0)),
        ],
        out_specs=pl.BlockSpec((blk, OUT_C), lambda i: (i, 0)),
        out_shape=jax.ShapeDtypeStruct((N_NODES, OUT_C), jnp.float32),
    )(psum, pcnt, h, bias2d)


NBUF = 8
AHEAD = 4


def _make_sc_aggregate(n_edges):
    assert n_edges % NW == 0
    epw = n_edges // NW
    assert epw % 8 == 0
    cpw = ((epw + K * NBUF - 1) // (K * NBUF)) * NBUF
    assert cpw >= 2 * NBUF
    n_groups = cpw // NBUF
    nfc, tail = divmod(epw, K)
    assert tail % 8 == 0 and epw % L == 0
    mesh = plsc.VectorSubcoreMesh(core_axis_name="c", subcore_axis_name="s")

    @functools.partial(
        pl.kernel,
        out_type=[
            jax.ShapeDtypeStruct((NC * SPAD, OUT_C), jnp.float32),
            jax.ShapeDtypeStruct((NC * SPAD,), jnp.float32),
        ],
        mesh=mesh,
        compiler_params=pltpu.CompilerParams(use_tc_tiling_on_sc=False),
        scratch_types=[
            pltpu.VMEM((cpw * K,), jnp.int32),
            pltpu.VMEM((cpw, K), jnp.int32),
            [pltpu.VMEM((K, OUT_C), jnp.float32) for _ in range(NBUF)],
            pltpu.VMEM((K,), jnp.float32),
            pltpu.VMEM((640,), jnp.float32),
            pltpu.VMEM_SHARED((SPAD, OUT_C), jnp.float32),
            pltpu.VMEM_SHARED((SPAD,), jnp.float32),
            pltpu.SemaphoreType.DMA,
            [pltpu.SemaphoreType.DMA for _ in range(NBUF)],
            [pltpu.SemaphoreType.DMA for _ in range(NBUF)],
            [pltpu.SemaphoreType.DMA for _ in range(NBUF)],
        ],
    )
    def sc_aggregate(src_hbm, dst_hbm, h_hbm, psum_hbm, pcnt_hbm,
                     src_all, dstp_all, rows, ones_v, zcnt_v,
                     ssum, scnt, stsem, gsems, ssems, csems):
        cid = lax.axis_index("c")
        sid = lax.axis_index("s")
        wid = sid * NC + cid

        zero16 = jnp.zeros((L,), jnp.float32)
        one16 = jnp.full((L,), 1.0, jnp.float32)
        lanes = lax.iota(jnp.int32, L)
        sub = K // L

        ebase = wid * epw
        pltpu.sync_copy(src_hbm.at[pl.ds(ebase, epw)],
                        src_all.at[pl.ds(0, epw)])

        def stage_dst(ch, _):
            pltpu.async_copy(dst_hbm.at[pl.ds(ebase + ch * K, K)],
                             dstp_all.at[ch], stsem)
            return 0
        lax.fori_loop(0, nfc, stage_dst, 0)
        if tail:
            pltpu.async_copy(dst_hbm.at[pl.ds(ebase + nfc * K, tail)],
                             dstp_all.at[nfc, pl.ds(0, tail)], stsem)

        def fill_ones(i, _):
            ones_v[pl.ds(i * L, L)] = one16
            return 0
        lax.fori_loop(0, K // L, fill_ones, 0)

        n_slots = cpw * K
        for t in range(epw // L, n_slots // L):
            src_all[pl.ds(t * L, L)] = (lanes * 613 + t * 97) % N_NODES
            ch, i = divmod(t, sub)
            dstp_all[ch, pl.ds(i * L, L)] = (
                TRASH + (t % (NTRASH // L)) * L + lanes)

        def drain_dst(ch, _):
            pltpu.make_async_copy(dst_hbm.at[pl.ds(0, K)],
                                  dstp_all.at[0], stsem).wait()
            return 0
        lax.fori_loop(0, nfc, drain_dst, 0)
        if tail:
            pltpu.make_async_copy(dst_hbm.at[pl.ds(0, tail)],
                                  dstp_all.at[0, pl.ds(0, tail)], stsem).wait()

        def mk(t, _):
            ch = t // sub
            i = t % sub
            sv = src_all[pl.ds(t * L, L)]
            dv = dstp_all[ch, pl.ds(i * L, L)]
            trash = TRASH + (t % (NTRASH // L)) * L + lanes
            dstp_all[ch, pl.ds(i * L, L)] = jnp.where(sv == dv, trash, dv)
            return 0
        lax.fori_loop(0, epw // L, mk, 0)

        zsub = OUT_C // L

        def zb(t, _):
            rows[0][t // zsub, pl.ds((t % zsub) * L, L)] = zero16
            return 0
        lax.fori_loop(0, K * zsub, zb, 0)

        def zc(i, _):
            zcnt_v[pl.ds(i * L, L)] = zero16
            return 0
        lax.fori_loop(0, 640 // L, zc, 0)

        rbase = sid * ROWS_PER_TILE
        nfull, rem = divmod(ROWS_PER_TILE, K)
        for t in range(nfull):
            pltpu.sync_copy(rows[0], ssum.at[pl.ds(rbase + t * K, K)])
        if rem:
            pltpu.sync_copy(rows[0].at[pl.ds(0, rem)],
                            ssum.at[pl.ds(rbase + nfull * K, rem)])
        pltpu.sync_copy(zcnt_v.at[pl.ds(0, ROWS_PER_TILE)],
                        scnt.at[pl.ds(rbase, ROWS_PER_TILE)])
        plsc.subcore_barrier()

        def issue_gather(ch, b):
            pltpu.async_copy(h_hbm.at[src_all.at[pl.ds(ch * K, K)]],
                             rows[b], gsems[b])

        def wait_gather(b):
            pltpu.make_async_copy(h_hbm.at[src_all.at[pl.ds(0, K)]], rows[b],
                                  gsems[b]).wait()

        def issue_scatters(ch, b):
            pltpu.async_copy(rows[b], ssum.at[dstp_all.at[ch]], ssems[b],
                             add=True)
            pltpu.async_copy(ones_v, scnt.at[dstp_all.at[ch]], csems[b],
                             add=True)

        def wait_scatters(b):
            pltpu.make_async_copy(rows[b], ssum.at[dstp_all.at[0]],
                                  ssems[b]).wait()
            pltpu.make_async_copy(ones_v, scnt.at[dstp_all.at[0]],
                                  csems[b]).wait()

        for b in range(AHEAD):
            issue_gather(b, b)

        for b in range(NBUF):
            bg = (b + AHEAD) % NBUF
            if b >= AHEAD:
                wait_scatters(bg)
            issue_gather(b + AHEAD, bg)
            wait_gather(b)
            issue_scatters(b, b)

        def group(g, _):
            base = g * NBUF
            for b in range(NBUF):
                bg = (b + AHEAD) % NBUF
                wait_scatters(bg)
                issue_gather(base + b + AHEAD, bg)
                wait_gather(b)
                issue_scatters(base + b, b)
            return 0
        lax.fori_loop(1, n_groups - 1, group, 0)

        tbase = (n_groups - 1) * NBUF
        for b in range(NBUF):
            if b < AHEAD:
                bg = (b + AHEAD) % NBUF
                wait_scatters(bg)
                issue_gather(tbase + b + AHEAD, bg)
            wait_gather(b)
            issue_scatters(tbase + b, b)

        for b in range(NBUF):
            wait_scatters(b)

        plsc.subcore_barrier()
        obase = cid * SPAD + rbase
        pltpu.sync_copy(ssum.at[pl.ds(rbase, ROWS_PER_TILE)],
                        psum_hbm.at[pl.ds(obase, ROWS_PER_TILE)])
        pltpu.sync_copy(scnt.at[pl.ds(rbase, ROWS_PER_TILE)],
                        pcnt_hbm.at[pl.ds(obase, ROWS_PER_TILE)])

    return sc_aggregate


def kernel(x, W, u, c, bias, adj):
    del u, c
    h = _matmul(x, W)

    src = adj[0]
    dst = adj[1]
    psum, pcnt = _make_sc_aggregate(src.shape[0])(src, dst, h)

    return _finalize(psum.reshape(NC, SPAD, OUT_C), pcnt.reshape(NC, SPAD, 1),
                     h, bias.reshape(1, OUT_C))


# speedup vs baseline: 23.3127x; 1.1983x over previous
import functools

import jax
import jax.numpy as jnp
from jax import lax
from jax.experimental import pallas as pl
from jax.experimental.pallas import tpu as pltpu
from jax.experimental.pallas import tpu_sc as plsc

N_NODES = 10000
IN_C = 128
OUT_C = 64

NC = 2
NS = 16
NW = NC * NS
L = 16

K = 128
ROWS_PER_TILE = 640
SPAD = ROWS_PER_TILE * NS
TRASH = N_NODES
NTRASH = SPAD - N_NODES
PSUM_W = 2 * OUT_C


def _mm_body(x_ref, w_ref, o_ref):
    o_ref[...] = jnp.dot(x_ref[...], w_ref[...],
                         preferred_element_type=jnp.float32)


def _matmul(x, W):
    blk = 1000
    grid = N_NODES // blk
    return pl.pallas_call(
        _mm_body,
        grid=(grid,),
        in_specs=[
            pl.BlockSpec((blk, IN_C), lambda i: (i, 0)),
            pl.BlockSpec((IN_C, OUT_C), lambda i: (0, 0)),
        ],
        out_specs=pl.BlockSpec((blk, OUT_C), lambda i: (i, 0)),
        out_shape=jax.ShapeDtypeStruct((N_NODES, OUT_C), jnp.float32),
    )(x, W)


def _fin_body(p0_ref, p1_ref, c0_ref, c1_ref, h_ref, b_ref, o_ref):
    s = p0_ref[:, :OUT_C] + p1_ref[:, :OUT_C] + h_ref[...]
    cnt = c0_ref[...] + c1_ref[...] + 1.0
    o_ref[...] = jnp.maximum(s / cnt[:, None] + b_ref[...], 0.0)


def _finalize(psum, pcnt, h, bias2d):
    blk = 1024
    grid = (N_NODES + blk - 1) // blk
    nb1 = SPAD // blk
    return pl.pallas_call(
        _fin_body,
        grid=(grid,),
        in_specs=[
            pl.BlockSpec((blk, PSUM_W), lambda i: (i, 0)),
            pl.BlockSpec((blk, PSUM_W), lambda i: (i + nb1, 0)),
            pl.BlockSpec((blk,), lambda i: (i,)),
            pl.BlockSpec((blk,), lambda i: (i + nb1,)),
            pl.BlockSpec((blk, OUT_C), lambda i: (i, 0)),
            pl.BlockSpec((1, OUT_C), lambda i: (0, 0)),
        ],
        out_specs=pl.BlockSpec((blk, OUT_C), lambda i: (i, 0)),
        out_shape=jax.ShapeDtypeStruct((N_NODES, OUT_C), jnp.float32),
    )(psum, psum, pcnt, pcnt, h, bias2d)


NBUF = 8
AHEAD = 4


def _make_sc_aggregate(n_edges):
    assert n_edges % NW == 0
    epw = n_edges // NW
    assert epw % 8 == 0
    cpw = ((epw + K * NBUF - 1) // (K * NBUF)) * NBUF
    assert cpw >= 2 * NBUF
    n_groups = cpw // NBUF
    nfc, tail = divmod(epw, K)
    assert tail % 8 == 0 and epw % L == 0
    mesh = plsc.VectorSubcoreMesh(core_axis_name="c", subcore_axis_name="s")

    @functools.partial(
        pl.kernel,
        out_type=[
            jax.ShapeDtypeStruct((NC * SPAD, PSUM_W), jnp.float32),
            jax.ShapeDtypeStruct((NC * SPAD,), jnp.float32),
        ],
        mesh=mesh,
        compiler_params=pltpu.CompilerParams(use_tc_tiling_on_sc=False),
        scratch_types=[
            pltpu.VMEM((cpw * K,), jnp.int32),
            pltpu.VMEM((cpw, K), jnp.int32),
            [pltpu.VMEM((K, OUT_C), jnp.float32) for _ in range(NBUF)],
            pltpu.VMEM((K,), jnp.float32),
            pltpu.VMEM((640,), jnp.float32),
            pltpu.VMEM_SHARED((SPAD, OUT_C), jnp.float32),
            pltpu.VMEM_SHARED((SPAD,), jnp.float32),
            pltpu.SemaphoreType.DMA,
            [pltpu.SemaphoreType.DMA for _ in range(NBUF)],
            [pltpu.SemaphoreType.DMA for _ in range(NBUF)],
            [pltpu.SemaphoreType.DMA for _ in range(NBUF)],
        ],
    )
    def sc_aggregate(src_hbm, dst_hbm, h_hbm, psum_hbm, pcnt_hbm,
                     src_all, dstp_all, rows, ones_v, zcnt_v,
                     ssum, scnt, stsem, gsems, ssems, csems):
        cid = lax.axis_index("c")
        sid = lax.axis_index("s")
        wid = sid * NC + cid

        zero16 = jnp.zeros((L,), jnp.float32)
        one16 = jnp.full((L,), 1.0, jnp.float32)
        lanes = lax.iota(jnp.int32, L)
        sub = K // L

        ebase = wid * epw
        pltpu.sync_copy(src_hbm.at[pl.ds(ebase, epw)],
                        src_all.at[pl.ds(0, epw)])

        def stage_dst(ch, _):
            pltpu.async_copy(dst_hbm.at[pl.ds(ebase + ch * K, K)],
                             dstp_all.at[ch], stsem)
            return 0
        lax.fori_loop(0, nfc, stage_dst, 0)
        if tail:
            pltpu.async_copy(dst_hbm.at[pl.ds(ebase + nfc * K, tail)],
                             dstp_all.at[nfc, pl.ds(0, tail)], stsem)

        def fill_ones(i, _):
            ones_v[pl.ds(i * L, L)] = one16
            return 0
        lax.fori_loop(0, K // L, fill_ones, 0)

        n_slots = cpw * K
        for t in range(epw // L, n_slots // L):
            src_all[pl.ds(t * L, L)] = (lanes * 613 + t * 97) % N_NODES
            ch, i = divmod(t, sub)
            dstp_all[ch, pl.ds(i * L, L)] = (
                TRASH + (t % (NTRASH // L)) * L + lanes)

        def drain_dst(ch, _):
            pltpu.make_async_copy(dst_hbm.at[pl.ds(0, K)],
                                  dstp_all.at[0], stsem).wait()
            return 0
        lax.fori_loop(0, nfc, drain_dst, 0)
        if tail:
            pltpu.make_async_copy(dst_hbm.at[pl.ds(0, tail)],
                                  dstp_all.at[0, pl.ds(0, tail)], stsem).wait()

        def mk(t, _):
            ch = t // sub
            i = t % sub
            sv = src_all[pl.ds(t * L, L)]
            dv = dstp_all[ch, pl.ds(i * L, L)]
            trash = TRASH + (t % (NTRASH // L)) * L + lanes
            dstp_all[ch, pl.ds(i * L, L)] = jnp.where(sv == dv, trash, dv)
            return 0
        lax.fori_loop(0, epw // L, mk, 0)

        zsub = OUT_C // L

        def zb(t, _):
            rows[0][t // zsub, pl.ds((t % zsub) * L, L)] = zero16
            return 0
        lax.fori_loop(0, K * zsub, zb, 0)

        def zc(i, _):
            zcnt_v[pl.ds(i * L, L)] = zero16
            return 0
        lax.fori_loop(0, 640 // L, zc, 0)

        rbase = sid * ROWS_PER_TILE
        nfull, rem = divmod(ROWS_PER_TILE, K)
        for t in range(nfull):
            pltpu.sync_copy(rows[0], ssum.at[pl.ds(rbase + t * K, K)])
        if rem:
            pltpu.sync_copy(rows[0].at[pl.ds(0, rem)],
                            ssum.at[pl.ds(rbase + nfull * K, rem)])
        pltpu.sync_copy(zcnt_v.at[pl.ds(0, ROWS_PER_TILE)],
                        scnt.at[pl.ds(rbase, ROWS_PER_TILE)])
        plsc.subcore_barrier()

        def issue_gather(ch, b):
            pltpu.async_copy(h_hbm.at[src_all.at[pl.ds(ch * K, K)]],
                             rows[b], gsems[b])

        def wait_gather(b):
            pltpu.make_async_copy(h_hbm.at[src_all.at[pl.ds(0, K)]], rows[b],
                                  gsems[b]).wait()

        def issue_scatters(ch, b):
            pltpu.async_copy(rows[b], ssum.at[dstp_all.at[ch]], ssems[b],
                             add=True)
            pltpu.async_copy(ones_v, scnt.at[dstp_all.at[ch]], csems[b],
                             add=True)

        def wait_scatters(b):
            pltpu.make_async_copy(rows[b], ssum.at[dstp_all.at[0]],
                                  ssems[b]).wait()
            pltpu.make_async_copy(ones_v, scnt.at[dstp_all.at[0]],
                                  csems[b]).wait()

        for b in range(AHEAD):
            issue_gather(b, b)

        for b in range(NBUF):
            bg = (b + AHEAD) % NBUF
            if b >= AHEAD:
                wait_scatters(bg)
            issue_gather(b + AHEAD, bg)
            wait_gather(b)
            issue_scatters(b, b)

        def group(g, _):
            base = g * NBUF
            for b in range(NBUF):
                bg = (b + AHEAD) % NBUF
                wait_scatters(bg)
                issue_gather(base + b + AHEAD, bg)
                wait_gather(b)
                issue_scatters(base + b, b)
            return 0
        lax.fori_loop(1, n_groups - 1, group, 0)

        tbase = (n_groups - 1) * NBUF
        for b in range(NBUF):
            if b < AHEAD:
                bg = (b + AHEAD) % NBUF
                wait_scatters(bg)
                issue_gather(tbase + b + AHEAD, bg)
            wait_gather(b)
            issue_scatters(tbase + b, b)

        for b in range(NBUF):
            wait_scatters(b)

        plsc.subcore_barrier()
        obase = cid * SPAD + rbase
        pltpu.sync_copy(ssum.at[pl.ds(rbase, ROWS_PER_TILE)],
                        psum_hbm.at[pl.ds(obase, ROWS_PER_TILE),
                                    pl.ds(0, OUT_C)])
        pltpu.sync_copy(scnt.at[pl.ds(rbase, ROWS_PER_TILE)],
                        pcnt_hbm.at[pl.ds(obase, ROWS_PER_TILE)])

    return sc_aggregate


def kernel(x, W, u, c, bias, adj):
    del u, c
    h = _matmul(x, W)

    src = adj[0]
    dst = adj[1]
    psum, pcnt = _make_sc_aggregate(src.shape[0])(src, dst, h)

    return _finalize(psum, pcnt, h, bias.reshape(1, OUT_C))


# speedup vs baseline: 23.6941x; 1.0164x over previous
import functools

import jax
import jax.numpy as jnp
from jax import lax
from jax.experimental import pallas as pl
from jax.experimental.pallas import tpu as pltpu
from jax.experimental.pallas import tpu_sc as plsc

N_NODES = 10000
IN_C = 128
OUT_C = 64

NC = 2
NS = 16
NW = NC * NS
L = 16

K = 128
ROWS_PER_TILE = 640
SPAD = ROWS_PER_TILE * NS
TRASH = N_NODES
NTRASH = SPAD - N_NODES
PSUM_W = 2 * OUT_C


def _mm_body(x_ref, w_ref, o_ref):
    o_ref[...] = jnp.dot(x_ref[...], w_ref[...],
                         preferred_element_type=jnp.float32)


def _matmul(x, W):
    blk = 1000
    grid = N_NODES // blk
    return pl.pallas_call(
        _mm_body,
        grid=(grid,),
        in_specs=[
            pl.BlockSpec((blk, IN_C), lambda i: (i, 0)),
            pl.BlockSpec((IN_C, OUT_C), lambda i: (0, 0)),
        ],
        out_specs=pl.BlockSpec((blk, OUT_C), lambda i: (i, 0)),
        out_shape=jax.ShapeDtypeStruct((N_NODES, OUT_C), jnp.float32),
    )(x, W)


def _fin_body(p0_ref, p1_ref, c0_ref, c1_ref, h_ref, b_ref, o_ref):
    s = p0_ref[:, :OUT_C] + p1_ref[:, :OUT_C] + h_ref[...]
    cnt = c0_ref[...] + c1_ref[...] + 1.0
    o_ref[...] = jnp.maximum(s / cnt[:, None] + b_ref[...], 0.0)


def _finalize(psum, pcnt, h, bias2d):
    blk = 1024
    grid = (N_NODES + blk - 1) // blk
    nb1 = SPAD // blk
    return pl.pallas_call(
        _fin_body,
        grid=(grid,),
        in_specs=[
            pl.BlockSpec((blk, PSUM_W), lambda i: (i, 0)),
            pl.BlockSpec((blk, PSUM_W), lambda i: (i + nb1, 0)),
            pl.BlockSpec((blk,), lambda i: (i,)),
            pl.BlockSpec((blk,), lambda i: (i + nb1,)),
            pl.BlockSpec((blk, OUT_C), lambda i: (i, 0)),
            pl.BlockSpec((1, OUT_C), lambda i: (0, 0)),
        ],
        out_specs=pl.BlockSpec((blk, OUT_C), lambda i: (i, 0)),
        out_shape=jax.ShapeDtypeStruct((N_NODES, OUT_C), jnp.float32),
    )(psum, psum, pcnt, pcnt, h, bias2d)


NBUF = 8
AHEAD = 6


def _make_sc_aggregate(n_edges):
    assert n_edges % NW == 0
    epw = n_edges // NW
    assert epw % 8 == 0
    cpw = ((epw + K * NBUF - 1) // (K * NBUF)) * NBUF
    assert cpw >= 2 * NBUF
    n_groups = cpw // NBUF
    nfc, tail = divmod(epw, K)
    assert tail % 8 == 0 and epw % L == 0
    mesh = plsc.VectorSubcoreMesh(core_axis_name="c", subcore_axis_name="s")

    @functools.partial(
        pl.kernel,
        out_type=[
            jax.ShapeDtypeStruct((NC * SPAD, PSUM_W), jnp.float32),
            jax.ShapeDtypeStruct((NC * SPAD,), jnp.float32),
        ],
        mesh=mesh,
        compiler_params=pltpu.CompilerParams(use_tc_tiling_on_sc=False),
        scratch_types=[
            pltpu.VMEM((cpw * K,), jnp.int32),
            pltpu.VMEM((cpw, K), jnp.int32),
            [pltpu.VMEM((K, OUT_C), jnp.float32) for _ in range(NBUF)],
            pltpu.VMEM((K,), jnp.float32),
            pltpu.VMEM((640,), jnp.float32),
            pltpu.VMEM_SHARED((SPAD, OUT_C), jnp.float32),
            pltpu.VMEM_SHARED((SPAD,), jnp.float32),
            pltpu.SemaphoreType.DMA,
            [pltpu.SemaphoreType.DMA for _ in range(NBUF)],
            [pltpu.SemaphoreType.DMA for _ in range(NBUF)],
            [pltpu.SemaphoreType.DMA for _ in range(NBUF)],
        ],
    )
    def sc_aggregate(src_hbm, dst_hbm, h_hbm, psum_hbm, pcnt_hbm,
                     src_all, dstp_all, rows, ones_v, zcnt_v,
                     ssum, scnt, stsem, gsems, ssems, csems):
        cid = lax.axis_index("c")
        sid = lax.axis_index("s")
        wid = sid * NC + cid

        zero16 = jnp.zeros((L,), jnp.float32)
        one16 = jnp.full((L,), 1.0, jnp.float32)
        lanes = lax.iota(jnp.int32, L)
        sub = K // L

        ebase = wid * epw
        pltpu.sync_copy(src_hbm.at[pl.ds(ebase, epw)],
                        src_all.at[pl.ds(0, epw)])

        def stage_dst(ch, _):
            pltpu.async_copy(dst_hbm.at[pl.ds(ebase + ch * K, K)],
                             dstp_all.at[ch], stsem)
            return 0
        lax.fori_loop(0, nfc, stage_dst, 0)
        if tail:
            pltpu.async_copy(dst_hbm.at[pl.ds(ebase + nfc * K, tail)],
                             dstp_all.at[nfc, pl.ds(0, tail)], stsem)

        def fill_ones(i, _):
            ones_v[pl.ds(i * L, L)] = one16
            return 0
        lax.fori_loop(0, K // L, fill_ones, 0)

        n_slots = cpw * K
        for t in range(epw // L, n_slots // L):
            src_all[pl.ds(t * L, L)] = (lanes * 613 + t * 97) % N_NODES
            ch, i = divmod(t, sub)
            dstp_all[ch, pl.ds(i * L, L)] = (
                TRASH + (t % (NTRASH // L)) * L + lanes)

        def drain_dst(ch, _):
            pltpu.make_async_copy(dst_hbm.at[pl.ds(0, K)],
                                  dstp_all.at[0], stsem).wait()
            return 0
        lax.fori_loop(0, nfc, drain_dst, 0)
        if tail:
            pltpu.make_async_copy(dst_hbm.at[pl.ds(0, tail)],
                                  dstp_all.at[0, pl.ds(0, tail)], stsem).wait()

        def mk(t, _):
            ch = t // sub
            i = t % sub
            sv = src_all[pl.ds(t * L, L)]
            dv = dstp_all[ch, pl.ds(i * L, L)]
            trash = TRASH + (t % (NTRASH // L)) * L + lanes
            dstp_all[ch, pl.ds(i * L, L)] = jnp.where(sv == dv, trash, dv)
            return 0
        lax.fori_loop(0, epw // L, mk, 0)

        zsub = OUT_C // L

        def zb(t, _):
            rows[0][t // zsub, pl.ds((t % zsub) * L, L)] = zero16
            return 0
        lax.fori_loop(0, K * zsub, zb, 0)

        def zc(i, _):
            zcnt_v[pl.ds(i * L, L)] = zero16
            return 0
        lax.fori_loop(0, 640 // L, zc, 0)

        rbase = sid * ROWS_PER_TILE
        nfull, rem = divmod(ROWS_PER_TILE, K)
        for t in range(nfull):
            pltpu.sync_copy(rows[0], ssum.at[pl.ds(rbase + t * K, K)])
        if rem:
            pltpu.sync_copy(rows[0].at[pl.ds(0, rem)],
                            ssum.at[pl.ds(rbase + nfull * K, rem)])
        pltpu.sync_copy(zcnt_v.at[pl.ds(0, ROWS_PER_TILE)],
                        scnt.at[pl.ds(rbase, ROWS_PER_TILE)])
        plsc.subcore_barrier()

        def issue_gather(ch, b):
            pltpu.async_copy(h_hbm.at[src_all.at[pl.ds(ch * K, K)]],
                             rows[b], gsems[b])

        def wait_gather(b):
            pltpu.make_async_copy(h_hbm.at[src_all.at[pl.ds(0, K)]], rows[b],
                                  gsems[b]).wait()

        def issue_scatters(ch, b):
            pltpu.async_copy(rows[b], ssum.at[dstp_all.at[ch]], ssems[b],
                             add=True)
            pltpu.async_copy(ones_v, scnt.at[dstp_all.at[ch]], csems[b],
                             add=True)

        def wait_scatters(b):
            pltpu.make_async_copy(rows[b], ssum.at[dstp_all.at[0]],
                                  ssems[b]).wait()
            pltpu.make_async_copy(ones_v, scnt.at[dstp_all.at[0]],
                                  csems[b]).wait()

        for b in range(AHEAD):
            issue_gather(b, b)

        for b in range(NBUF):
            bg = (b + AHEAD) % NBUF
            if b >= NBUF - AHEAD:
                wait_scatters(bg)
            issue_gather(b + AHEAD, bg)
            wait_gather(b)
            issue_scatters(b, b)

        def group(g, _):
            base = g * NBUF
            for b in range(NBUF):
                bg = (b + AHEAD) % NBUF
                wait_scatters(bg)
                issue_gather(base + b + AHEAD, bg)
                wait_gather(b)
                issue_scatters(base + b, b)
            return 0
        lax.fori_loop(1, n_groups - 1, group, 0)

        tbase = (n_groups - 1) * NBUF
        for b in range(NBUF):
            if b < NBUF - AHEAD:
                bg = (b + AHEAD) % NBUF
                wait_scatters(bg)
                issue_gather(tbase + b + AHEAD, bg)
            wait_gather(b)
            issue_scatters(tbase + b, b)

        for b in range(NBUF):
            wait_scatters(b)

        plsc.subcore_barrier()
        obase = cid * SPAD + rbase
        pltpu.sync_copy(ssum.at[pl.ds(rbase, ROWS_PER_TILE)],
                        psum_hbm.at[pl.ds(obase, ROWS_PER_TILE),
                                    pl.ds(0, OUT_C)])
        pltpu.sync_copy(scnt.at[pl.ds(rbase, ROWS_PER_TILE)],
                        pcnt_hbm.at[pl.ds(obase, ROWS_PER_TILE)])

    return sc_aggregate


def kernel(x, W, u, c, bias, adj):
    del u, c
    h = _matmul(x, W)

    src = adj[0]
    dst = adj[1]
    psum, pcnt = _make_sc_aggregate(src.shape[0])(src, dst, h)

    return _finalize(psum, pcnt, h, bias.reshape(1, OUT_C))


# speedup vs baseline: 25.8158x; 1.0895x over previous
import functools

import jax
import jax.numpy as jnp
from jax import lax
from jax.experimental import pallas as pl
from jax.experimental.pallas import tpu as pltpu
from jax.experimental.pallas import tpu_sc as plsc

N_NODES = 10000
IN_C = 128
OUT_C = 64

NC = 2
NS = 16
NW = NC * NS
L = 16

K = 128
ROWS_PER_TILE = 640
SPAD = ROWS_PER_TILE * NS
TRASH = N_NODES
NTRASH = SPAD - N_NODES
PSUM_W = 2 * OUT_C


def _mm_body(x_ref, w_ref, o_ref):
    o_ref[...] = jnp.dot(x_ref[...], w_ref[...],
                         preferred_element_type=jnp.float32)


def _matmul(x, W):
    blk = 1000
    grid = N_NODES // blk
    return pl.pallas_call(
        _mm_body,
        grid=(grid,),
        in_specs=[
            pl.BlockSpec((blk, IN_C), lambda i: (i, 0)),
            pl.BlockSpec((IN_C, OUT_C), lambda i: (0, 0)),
        ],
        out_specs=pl.BlockSpec((blk, OUT_C), lambda i: (i, 0)),
        out_shape=jax.ShapeDtypeStruct((N_NODES, OUT_C), jnp.float32),
    )(x, W)


def _fin_body(p0_ref, p1_ref, c0_ref, c1_ref, h_ref, b_ref, o_ref):
    s = p0_ref[:, :OUT_C] + p1_ref[:, :OUT_C] + h_ref[...]
    cnt = c0_ref[...] + c1_ref[...] + 1.0
    o_ref[...] = jnp.maximum(s / cnt[:, None] + b_ref[...], 0.0)


def _finalize(psum, pcnt, h, bias2d):
    blk = 1024
    grid = (N_NODES + blk - 1) // blk
    nb1 = SPAD // blk
    return pl.pallas_call(
        _fin_body,
        grid=(grid,),
        in_specs=[
            pl.BlockSpec((blk, PSUM_W), lambda i: (i, 0)),
            pl.BlockSpec((blk, PSUM_W), lambda i: (i + nb1, 0)),
            pl.BlockSpec((blk,), lambda i: (i,)),
            pl.BlockSpec((blk,), lambda i: (i + nb1,)),
            pl.BlockSpec((blk, OUT_C), lambda i: (i, 0)),
            pl.BlockSpec((1, OUT_C), lambda i: (0, 0)),
        ],
        out_specs=pl.BlockSpec((blk, OUT_C), lambda i: (i, 0)),
        out_shape=jax.ShapeDtypeStruct((N_NODES, OUT_C), jnp.float32),
    )(psum, psum, pcnt, pcnt, h, bias2d)


NBUF = 8
AHEAD = 6


def _make_sc_aggregate(n_edges):
    assert n_edges % NW == 0
    epw = n_edges // NW
    assert epw % 8 == 0
    cpw = ((epw + K * NBUF - 1) // (K * NBUF)) * NBUF
    assert cpw >= 2 * NBUF
    n_groups = cpw // NBUF
    nfc, tail = divmod(epw, K)
    assert tail % 8 == 0 and epw % L == 0
    mesh = plsc.VectorSubcoreMesh(core_axis_name="c", subcore_axis_name="s")

    @functools.partial(
        pl.kernel,
        out_type=[
            jax.ShapeDtypeStruct((NC * SPAD, PSUM_W), jnp.float32),
            jax.ShapeDtypeStruct((NC * SPAD,), jnp.float32),
        ],
        mesh=mesh,
        compiler_params=pltpu.CompilerParams(use_tc_tiling_on_sc=False),
        scratch_types=[
            pltpu.VMEM((cpw * K,), jnp.int32),
            pltpu.VMEM((cpw, K), jnp.int32),
            [pltpu.VMEM((K, OUT_C), jnp.float32) for _ in range(NBUF)],
            pltpu.VMEM((K,), jnp.float32),
            pltpu.VMEM((640,), jnp.float32),
            pltpu.VMEM_SHARED((SPAD, OUT_C), jnp.float32),
            pltpu.VMEM_SHARED((SPAD,), jnp.float32),
            pltpu.SemaphoreType.DMA,
            [pltpu.SemaphoreType.DMA for _ in range(NBUF)],
            [pltpu.SemaphoreType.DMA for _ in range(NBUF)],
            [pltpu.SemaphoreType.DMA for _ in range(NBUF)],
        ],
    )
    def sc_aggregate(adj_hbm, h_hbm, psum_hbm, pcnt_hbm,
                     src_all, dstp_all, rows, ones_v, zcnt_v,
                     ssum, scnt, stsem, gsems, ssems, csems):
        src_hbm = adj_hbm.at[0]
        dst_hbm = adj_hbm.at[1]
        cid = lax.axis_index("c")
        sid = lax.axis_index("s")
        wid = sid * NC + cid

        zero16 = jnp.zeros((L,), jnp.float32)
        one16 = jnp.full((L,), 1.0, jnp.float32)
        lanes = lax.iota(jnp.int32, L)
        sub = K // L

        ebase = wid * epw
        pltpu.sync_copy(src_hbm.at[pl.ds(ebase, epw)],
                        src_all.at[pl.ds(0, epw)])

        def stage_dst(ch, _):
            pltpu.async_copy(dst_hbm.at[pl.ds(ebase + ch * K, K)],
                             dstp_all.at[ch], stsem)
            return 0
        lax.fori_loop(0, nfc, stage_dst, 0)
        if tail:
            pltpu.async_copy(dst_hbm.at[pl.ds(ebase + nfc * K, tail)],
                             dstp_all.at[nfc, pl.ds(0, tail)], stsem)

        def fill_ones(i, _):
            ones_v[pl.ds(i * L, L)] = one16
            return 0
        lax.fori_loop(0, K // L, fill_ones, 0)

        n_slots = cpw * K
        for t in range(epw // L, n_slots // L):
            src_all[pl.ds(t * L, L)] = (lanes * 613 + t * 97) % N_NODES
            ch, i = divmod(t, sub)
            dstp_all[ch, pl.ds(i * L, L)] = (
                TRASH + (t % (NTRASH // L)) * L + lanes)

        def drain_dst(ch, _):
            pltpu.make_async_copy(dst_hbm.at[pl.ds(0, K)],
                                  dstp_all.at[0], stsem).wait()
            return 0
        lax.fori_loop(0, nfc, drain_dst, 0)
        if tail:
            pltpu.make_async_copy(dst_hbm.at[pl.ds(0, tail)],
                                  dstp_all.at[0, pl.ds(0, tail)], stsem).wait()

        def mk(t, _):
            ch = t // sub
            i = t % sub
            sv = src_all[pl.ds(t * L, L)]
            dv = dstp_all[ch, pl.ds(i * L, L)]
            trash = TRASH + (t % (NTRASH // L)) * L + lanes
            dstp_all[ch, pl.ds(i * L, L)] = jnp.where(sv == dv, trash, dv)
            return 0
        lax.fori_loop(0, epw // L, mk, 0)

        zsub = OUT_C // L

        def zb(t, _):
            rows[0][t // zsub, pl.ds((t % zsub) * L, L)] = zero16
            return 0
        lax.fori_loop(0, K * zsub, zb, 0)

        def zc(i, _):
            zcnt_v[pl.ds(i * L, L)] = zero16
            return 0
        lax.fori_loop(0, 640 // L, zc, 0)

        rbase = sid * ROWS_PER_TILE
        nfull, rem = divmod(ROWS_PER_TILE, K)
        for t in range(nfull):
            pltpu.sync_copy(rows[0], ssum.at[pl.ds(rbase + t * K, K)])
        if rem:
            pltpu.sync_copy(rows[0].at[pl.ds(0, rem)],
                            ssum.at[pl.ds(rbase + nfull * K, rem)])
        pltpu.sync_copy(zcnt_v.at[pl.ds(0, ROWS_PER_TILE)],
                        scnt.at[pl.ds(rbase, ROWS_PER_TILE)])
        plsc.subcore_barrier()

        def issue_gather(ch, b):
            pltpu.async_copy(h_hbm.at[src_all.at[pl.ds(ch * K, K)]],
                             rows[b], gsems[b])

        def wait_gather(b):
            pltpu.make_async_copy(h_hbm.at[src_all.at[pl.ds(0, K)]], rows[b],
                                  gsems[b]).wait()

        def issue_scatters(ch, b):
            pltpu.async_copy(rows[b], ssum.at[dstp_all.at[ch]], ssems[b],
                             add=True)
            pltpu.async_copy(ones_v, scnt.at[dstp_all.at[ch]], csems[b],
                             add=True)

        def wait_scatters(b):
            pltpu.make_async_copy(rows[b], ssum.at[dstp_all.at[0]],
                                  ssems[b]).wait()
            pltpu.make_async_copy(ones_v, scnt.at[dstp_all.at[0]],
                                  csems[b]).wait()

        for b in range(AHEAD):
            issue_gather(b, b)

        for b in range(NBUF):
            bg = (b + AHEAD) % NBUF
            if b >= NBUF - AHEAD:
                wait_scatters(bg)
            issue_gather(b + AHEAD, bg)
            wait_gather(b)
            issue_scatters(b, b)

        def group(g, _):
            base = g * NBUF
            for b in range(NBUF):
                bg = (b + AHEAD) % NBUF
                wait_scatters(bg)
                issue_gather(base + b + AHEAD, bg)
                wait_gather(b)
                issue_scatters(base + b, b)
            return 0
        lax.fori_loop(1, n_groups - 1, group, 0)

        tbase = (n_groups - 1) * NBUF
        for b in range(NBUF):
            if b < NBUF - AHEAD:
                bg = (b + AHEAD) % NBUF
                wait_scatters(bg)
                issue_gather(tbase + b + AHEAD, bg)
            wait_gather(b)
            issue_scatters(tbase + b, b)

        for b in range(NBUF):
            wait_scatters(b)

        plsc.subcore_barrier()
        obase = cid * SPAD + rbase
        pltpu.sync_copy(ssum.at[pl.ds(rbase, ROWS_PER_TILE)],
                        psum_hbm.at[pl.ds(obase, ROWS_PER_TILE),
                                    pl.ds(0, OUT_C)])
        pltpu.sync_copy(scnt.at[pl.ds(rbase, ROWS_PER_TILE)],
                        pcnt_hbm.at[pl.ds(obase, ROWS_PER_TILE)])

    return sc_aggregate


def kernel(x, W, u, c, bias, adj):
    del u, c
    h = _matmul(x, W)

    psum, pcnt = _make_sc_aggregate(adj.shape[1])(adj, h)

    return _finalize(psum, pcnt, h, bias.reshape(1, OUT_C))


# speedup vs baseline: 26.5081x; 1.0268x over previous
import functools

import jax
import jax.numpy as jnp
from jax import lax
from jax.experimental import pallas as pl
from jax.experimental.pallas import tpu as pltpu
from jax.experimental.pallas import tpu_sc as plsc

N_NODES = 10000
IN_C = 128
OUT_C = 64

NC = 2
NS = 16
NW = NC * NS
L = 16

K = 128
ROWS_PER_TILE = 640
SPAD = ROWS_PER_TILE * NS
TRASH = N_NODES
NTRASH = SPAD - N_NODES
PSUM_W = 2 * OUT_C


def _mm_body(x_ref, w_ref, o_ref):
    o_ref[...] = jnp.dot(x_ref[...], w_ref[...],
                         preferred_element_type=jnp.float32)


def _matmul(x, W):
    blk = 2000
    grid = N_NODES // blk
    return pl.pallas_call(
        _mm_body,
        grid=(grid,),
        in_specs=[
            pl.BlockSpec((blk, IN_C), lambda i: (i, 0)),
            pl.BlockSpec((IN_C, OUT_C), lambda i: (0, 0)),
        ],
        out_specs=pl.BlockSpec((blk, OUT_C), lambda i: (i, 0)),
        out_shape=jax.ShapeDtypeStruct((N_NODES, OUT_C), jnp.float32),
    )(x, W)


def _fin_body(p0_ref, p1_ref, c0_ref, c1_ref, h_ref, b_ref, o_ref):
    s = p0_ref[:, :OUT_C] + p1_ref[:, :OUT_C] + h_ref[...]
    cnt = c0_ref[...] + c1_ref[...] + 1.0
    o_ref[...] = jnp.maximum(s / cnt[:, None] + b_ref[...], 0.0)


def _finalize(psum, pcnt, h, bias2d):
    blk = 1024
    grid = (N_NODES + blk - 1) // blk
    nb1 = SPAD // blk
    return pl.pallas_call(
        _fin_body,
        grid=(grid,),
        in_specs=[
            pl.BlockSpec((blk, PSUM_W), lambda i: (i, 0)),
            pl.BlockSpec((blk, PSUM_W), lambda i: (i + nb1, 0)),
            pl.BlockSpec((blk,), lambda i: (i,)),
            pl.BlockSpec((blk,), lambda i: (i + nb1,)),
            pl.BlockSpec((blk, OUT_C), lambda i: (i, 0)),
            pl.BlockSpec((1, OUT_C), lambda i: (0, 0)),
        ],
        out_specs=pl.BlockSpec((blk, OUT_C), lambda i: (i, 0)),
        out_shape=jax.ShapeDtypeStruct((N_NODES, OUT_C), jnp.float32),
    )(psum, psum, pcnt, pcnt, h, bias2d)


NBUF = 8
AHEAD = 6


def _make_sc_aggregate(n_edges):
    assert n_edges % NW == 0
    epw = n_edges // NW
    assert epw % 8 == 0
    cpw = ((epw + K * NBUF - 1) // (K * NBUF)) * NBUF
    assert cpw >= 2 * NBUF
    n_groups = cpw // NBUF
    nfc, tail = divmod(epw, K)
    assert tail % 8 == 0 and epw % L == 0
    mesh = plsc.VectorSubcoreMesh(core_axis_name="c", subcore_axis_name="s")

    @functools.partial(
        pl.kernel,
        out_type=[
            jax.ShapeDtypeStruct((NC * SPAD, PSUM_W), jnp.float32),
            jax.ShapeDtypeStruct((NC * SPAD,), jnp.float32),
        ],
        mesh=mesh,
        compiler_params=pltpu.CompilerParams(use_tc_tiling_on_sc=False),
        scratch_types=[
            pltpu.VMEM((cpw * K,), jnp.int32),
            pltpu.VMEM((cpw, K), jnp.int32),
            [pltpu.VMEM((K, OUT_C), jnp.float32) for _ in range(NBUF)],
            pltpu.VMEM((K,), jnp.float32),
            pltpu.VMEM((640,), jnp.float32),
            pltpu.VMEM_SHARED((SPAD, OUT_C), jnp.float32),
            pltpu.VMEM_SHARED((SPAD,), jnp.float32),
            pltpu.SemaphoreType.DMA,
            [pltpu.SemaphoreType.DMA for _ in range(NBUF)],
            [pltpu.SemaphoreType.DMA for _ in range(NBUF)],
            [pltpu.SemaphoreType.DMA for _ in range(NBUF)],
        ],
    )
    def sc_aggregate(adj_hbm, h_hbm, psum_hbm, pcnt_hbm,
                     src_all, dstp_all, rows, ones_v, zcnt_v,
                     ssum, scnt, stsem, gsems, ssems, csems):
        src_hbm = adj_hbm.at[0]
        dst_hbm = adj_hbm.at[1]
        cid = lax.axis_index("c")
        sid = lax.axis_index("s")
        wid = sid * NC + cid

        zero16 = jnp.zeros((L,), jnp.float32)
        one16 = jnp.full((L,), 1.0, jnp.float32)
        lanes = lax.iota(jnp.int32, L)
        sub = K // L

        ebase = wid * epw
        src_stage_desc = pltpu.async_copy(src_hbm.at[pl.ds(ebase, epw)],
                                          src_all.at[pl.ds(0, epw)], stsem)

        def stage_dst(ch, _):
            pltpu.async_copy(dst_hbm.at[pl.ds(ebase + ch * K, K)],
                             dstp_all.at[ch], stsem)
            return 0
        lax.fori_loop(0, nfc, stage_dst, 0)
        if tail:
            pltpu.async_copy(dst_hbm.at[pl.ds(ebase + nfc * K, tail)],
                             dstp_all.at[nfc, pl.ds(0, tail)], stsem)

        def fill_ones(i, _):
            ones_v[pl.ds(i * L, L)] = one16
            return 0
        lax.fori_loop(0, K // L, fill_ones, 0)

        n_slots = cpw * K
        for t in range(epw // L, n_slots // L):
            src_all[pl.ds(t * L, L)] = (lanes * 613 + t * 97) % N_NODES
            ch, i = divmod(t, sub)
            dstp_all[ch, pl.ds(i * L, L)] = (
                TRASH + (t % (NTRASH // L)) * L + lanes)

        src_stage_desc.wait()

        def drain_dst(ch, _):
            pltpu.make_async_copy(dst_hbm.at[pl.ds(0, K)],
                                  dstp_all.at[0], stsem).wait()
            return 0
        lax.fori_loop(0, nfc, drain_dst, 0)
        if tail:
            pltpu.make_async_copy(dst_hbm.at[pl.ds(0, tail)],
                                  dstp_all.at[0, pl.ds(0, tail)], stsem).wait()

        def mk(t, _):
            ch = t // sub
            i = t % sub
            sv = src_all[pl.ds(t * L, L)]
            dv = dstp_all[ch, pl.ds(i * L, L)]
            trash = TRASH + (t % (NTRASH // L)) * L + lanes
            dstp_all[ch, pl.ds(i * L, L)] = jnp.where(sv == dv, trash, dv)
            return 0
        lax.fori_loop(0, epw // L, mk, 0)

        zsub = OUT_C // L

        def zb(t, _):
            rows[0][t // zsub, pl.ds((t % zsub) * L, L)] = zero16
            return 0
        lax.fori_loop(0, K * zsub, zb, 0)

        def zc(i, _):
            zcnt_v[pl.ds(i * L, L)] = zero16
            return 0
        lax.fori_loop(0, 640 // L, zc, 0)

        rbase = sid * ROWS_PER_TILE
        nfull, rem = divmod(ROWS_PER_TILE, K)
        for t in range(nfull):
            pltpu.sync_copy(rows[0], ssum.at[pl.ds(rbase + t * K, K)])
        if rem:
            pltpu.sync_copy(rows[0].at[pl.ds(0, rem)],
                            ssum.at[pl.ds(rbase + nfull * K, rem)])
        pltpu.sync_copy(zcnt_v.at[pl.ds(0, ROWS_PER_TILE)],
                        scnt.at[pl.ds(rbase, ROWS_PER_TILE)])
        plsc.subcore_barrier()

        def issue_gather(ch, b):
            pltpu.async_copy(h_hbm.at[src_all.at[pl.ds(ch * K, K)]],
                             rows[b], gsems[b])

        def wait_gather(b):
            pltpu.make_async_copy(h_hbm.at[src_all.at[pl.ds(0, K)]], rows[b],
                                  gsems[b]).wait()

        def issue_scatters(ch, b):
            pltpu.async_copy(rows[b], ssum.at[dstp_all.at[ch]], ssems[b],
                             add=True)
            pltpu.async_copy(ones_v, scnt.at[dstp_all.at[ch]], csems[b],
                             add=True)

        def wait_scatters(b):
            pltpu.make_async_copy(rows[b], ssum.at[dstp_all.at[0]],
                                  ssems[b]).wait()
            pltpu.make_async_copy(ones_v, scnt.at[dstp_all.at[0]],
                                  csems[b]).wait()

        for b in range(AHEAD):
            issue_gather(b, b)

        for b in range(NBUF):
            bg = (b + AHEAD) % NBUF
            if b >= NBUF - AHEAD:
                wait_scatters(bg)
            issue_gather(b + AHEAD, bg)
            wait_gather(b)
            issue_scatters(b, b)

        def group(g, _):
            base = g * NBUF
            for b in range(NBUF):
                bg = (b + AHEAD) % NBUF
                wait_scatters(bg)
                issue_gather(base + b + AHEAD, bg)
                wait_gather(b)
                issue_scatters(base + b, b)
            return 0
        lax.fori_loop(1, n_groups - 1, group, 0)

        tbase = (n_groups - 1) * NBUF
        for b in range(NBUF):
            if b < NBUF - AHEAD:
                bg = (b + AHEAD) % NBUF
                wait_scatters(bg)
                issue_gather(tbase + b + AHEAD, bg)
            wait_gather(b)
            issue_scatters(tbase + b, b)

        for b in range(NBUF):
            wait_scatters(b)

        plsc.subcore_barrier()
        obase = cid * SPAD + rbase
        pltpu.sync_copy(ssum.at[pl.ds(rbase, ROWS_PER_TILE)],
                        psum_hbm.at[pl.ds(obase, ROWS_PER_TILE),
                                    pl.ds(0, OUT_C)])
        pltpu.sync_copy(scnt.at[pl.ds(rbase, ROWS_PER_TILE)],
                        pcnt_hbm.at[pl.ds(obase, ROWS_PER_TILE)])

    return sc_aggregate


def kernel(x, W, u, c, bias, adj):
    del u, c
    h = _matmul(x, W)

    psum, pcnt = _make_sc_aggregate(adj.shape[1])(adj, h)

    return _finalize(psum, pcnt, h, bias.reshape(1, OUT_C))


# speedup vs baseline: 27.9138x; 1.0530x over previous
import functools

import jax
import jax.numpy as jnp
from jax import lax
from jax.experimental import pallas as pl
from jax.experimental.pallas import tpu as pltpu
from jax.experimental.pallas import tpu_sc as plsc

N_NODES = 10000
IN_C = 128
OUT_C = 64

NC = 2
NS = 16
NW = NC * NS
L = 16

K = 128
ROWS_PER_TILE = 640
SPAD = ROWS_PER_TILE * NS
TRASH = N_NODES
NTRASH = SPAD - N_NODES
PSUM_W = 2 * OUT_C


def _mm_body(x_ref, w_ref, o_ref):
    o_ref[...] = jnp.dot(x_ref[...], w_ref[...],
                         preferred_element_type=jnp.float32)


def _matmul(x, W):
    blk = 2000
    grid = N_NODES // blk
    return pl.pallas_call(
        _mm_body,
        grid=(grid,),
        in_specs=[
            pl.BlockSpec((blk, IN_C), lambda i: (i, 0)),
            pl.BlockSpec((IN_C, OUT_C), lambda i: (0, 0)),
        ],
        out_specs=pl.BlockSpec((blk, OUT_C), lambda i: (i, 0)),
        out_shape=jax.ShapeDtypeStruct((N_NODES, OUT_C), jnp.float32),
    )(x, W)


def _fin_body(p0_ref, p1_ref, c0_ref, c1_ref, h_ref, b_ref, o_ref):
    s = p0_ref[:, :OUT_C] + p1_ref[:, :OUT_C] + h_ref[...]
    cnt = c0_ref[...] + c1_ref[...] + 1.0
    res = jnp.maximum(s / cnt[:, None] + b_ref[...], 0.0)
    o_ref[...] = res.T


def _finalize(psum, pcnt, h, bias2d):
    blk = 1024
    grid = (N_NODES + blk - 1) // blk
    nb1 = SPAD // blk
    return pl.pallas_call(
        _fin_body,
        grid=(grid,),
        in_specs=[
            pl.BlockSpec((blk, PSUM_W), lambda i: (i, 0)),
            pl.BlockSpec((blk, PSUM_W), lambda i: (i + nb1, 0)),
            pl.BlockSpec((blk,), lambda i: (i,)),
            pl.BlockSpec((blk,), lambda i: (i + nb1,)),
            pl.BlockSpec((blk, OUT_C), lambda i: (i, 0)),
            pl.BlockSpec((1, OUT_C), lambda i: (0, 0)),
        ],
        out_specs=pl.BlockSpec((OUT_C, blk), lambda i: (0, i)),
        out_shape=jax.ShapeDtypeStruct((OUT_C, N_NODES), jnp.float32),
    )(psum, psum, pcnt, pcnt, h, bias2d).T


NBUF = 8
AHEAD = 6


def _make_sc_aggregate(n_edges):
    assert n_edges % NW == 0
    epw = n_edges // NW
    assert epw % 8 == 0
    cpw = ((epw + K * NBUF - 1) // (K * NBUF)) * NBUF
    assert cpw >= 2 * NBUF
    n_groups = cpw // NBUF
    nfc, tail = divmod(epw, K)
    assert tail % 8 == 0 and epw % L == 0
    mesh = plsc.VectorSubcoreMesh(core_axis_name="c", subcore_axis_name="s")

    @functools.partial(
        pl.kernel,
        out_type=[
            jax.ShapeDtypeStruct((NC * SPAD, PSUM_W), jnp.float32),
            jax.ShapeDtypeStruct((NC * SPAD,), jnp.float32),
        ],
        mesh=mesh,
        compiler_params=pltpu.CompilerParams(use_tc_tiling_on_sc=False),
        scratch_types=[
            pltpu.VMEM((cpw * K,), jnp.int32),
            pltpu.VMEM((cpw, K), jnp.int32),
            [pltpu.VMEM((K, OUT_C), jnp.float32) for _ in range(NBUF)],
            pltpu.VMEM((K,), jnp.float32),
            pltpu.VMEM((640,), jnp.float32),
            pltpu.VMEM_SHARED((SPAD, OUT_C), jnp.float32),
            pltpu.VMEM_SHARED((SPAD,), jnp.float32),
            pltpu.SemaphoreType.DMA,
            [pltpu.SemaphoreType.DMA for _ in range(NBUF)],
            [pltpu.SemaphoreType.DMA for _ in range(NBUF)],
            [pltpu.SemaphoreType.DMA for _ in range(NBUF)],
        ],
    )
    def sc_aggregate(adj_hbm, h_hbm, psum_hbm, pcnt_hbm,
                     src_all, dstp_all, rows, ones_v, zcnt_v,
                     ssum, scnt, stsem, gsems, ssems, csems):
        src_hbm = adj_hbm.at[0]
        dst_hbm = adj_hbm.at[1]
        cid = lax.axis_index("c")
        sid = lax.axis_index("s")
        wid = sid * NC + cid

        zero16 = jnp.zeros((L,), jnp.float32)
        one16 = jnp.full((L,), 1.0, jnp.float32)
        lanes = lax.iota(jnp.int32, L)
        sub = K // L

        ebase = wid * epw
        src_stage_desc = pltpu.async_copy(src_hbm.at[pl.ds(ebase, epw)],
                                          src_all.at[pl.ds(0, epw)], stsem)

        def stage_dst(ch, _):
            pltpu.async_copy(dst_hbm.at[pl.ds(ebase + ch * K, K)],
                             dstp_all.at[ch], stsem)
            return 0
        lax.fori_loop(0, nfc, stage_dst, 0)
        if tail:
            pltpu.async_copy(dst_hbm.at[pl.ds(ebase + nfc * K, tail)],
                             dstp_all.at[nfc, pl.ds(0, tail)], stsem)

        def fill_ones(i, _):
            ones_v[pl.ds(i * L, L)] = one16
            return 0
        lax.fori_loop(0, K // L, fill_ones, 0)

        n_slots = cpw * K
        for t in range(epw // L, n_slots // L):
            src_all[pl.ds(t * L, L)] = (lanes * 613 + t * 97) % N_NODES
            ch, i = divmod(t, sub)
            dstp_all[ch, pl.ds(i * L, L)] = (
                TRASH + (t % (NTRASH // L)) * L + lanes)

        src_stage_desc.wait()

        def drain_dst(ch, _):
            pltpu.make_async_copy(dst_hbm.at[pl.ds(0, K)],
                                  dstp_all.at[0], stsem).wait()
            return 0
        lax.fori_loop(0, nfc, drain_dst, 0)
        if tail:
            pltpu.make_async_copy(dst_hbm.at[pl.ds(0, tail)],
                                  dstp_all.at[0, pl.ds(0, tail)], stsem).wait()

        def mk(t, _):
            ch = t // sub
            i = t % sub
            sv = src_all[pl.ds(t * L, L)]
            dv = dstp_all[ch, pl.ds(i * L, L)]
            trash = TRASH + (t % (NTRASH // L)) * L + lanes
            dstp_all[ch, pl.ds(i * L, L)] = jnp.where(sv == dv, trash, dv)
            return 0
        lax.fori_loop(0, epw // L, mk, 0)

        zsub = OUT_C // L

        def zb(t, _):
            rows[0][t // zsub, pl.ds((t % zsub) * L, L)] = zero16
            return 0
        lax.fori_loop(0, K * zsub, zb, 0)

        def zc(i, _):
            zcnt_v[pl.ds(i * L, L)] = zero16
            return 0
        lax.fori_loop(0, 640 // L, zc, 0)

        rbase = sid * ROWS_PER_TILE
        nfull, rem = divmod(ROWS_PER_TILE, K)
        for t in range(nfull):
            pltpu.sync_copy(rows[0], ssum.at[pl.ds(rbase + t * K, K)])
        if rem:
            pltpu.sync_copy(rows[0].at[pl.ds(0, rem)],
                            ssum.at[pl.ds(rbase + nfull * K, rem)])
        pltpu.sync_copy(zcnt_v.at[pl.ds(0, ROWS_PER_TILE)],
                        scnt.at[pl.ds(rbase, ROWS_PER_TILE)])
        plsc.subcore_barrier()

        def issue_gather(ch, b):
            pltpu.async_copy(h_hbm.at[src_all.at[pl.ds(ch * K, K)]],
                             rows[b], gsems[b])

        def wait_gather(b):
            pltpu.make_async_copy(h_hbm.at[src_all.at[pl.ds(0, K)]], rows[b],
                                  gsems[b]).wait()

        def issue_scatters(ch, b):
            pltpu.async_copy(rows[b], ssum.at[dstp_all.at[ch]], ssems[b],
                             add=True)
            pltpu.async_copy(ones_v, scnt.at[dstp_all.at[ch]], csems[b],
                             add=True)

        def wait_scatters(b):
            pltpu.make_async_copy(rows[b], ssum.at[dstp_all.at[0]],
                                  ssems[b]).wait()
            pltpu.make_async_copy(ones_v, scnt.at[dstp_all.at[0]],
                                  csems[b]).wait()

        for b in range(AHEAD):
            issue_gather(b, b)

        for b in range(NBUF):
            bg = (b + AHEAD) % NBUF
            if b >= NBUF - AHEAD:
                wait_scatters(bg)
            issue_gather(b + AHEAD, bg)
            wait_gather(b)
            issue_scatters(b, b)

        def group(g, _):
            base = g * NBUF
            for b in range(NBUF):
                bg = (b + AHEAD) % NBUF
                wait_scatters(bg)
                issue_gather(base + b + AHEAD, bg)
                wait_gather(b)
                issue_scatters(base + b, b)
            return 0
        lax.fori_loop(1, n_groups - 1, group, 0)

        tbase = (n_groups - 1) * NBUF
        for b in range(NBUF):
            if b < NBUF - AHEAD:
                bg = (b + AHEAD) % NBUF
                wait_scatters(bg)
                issue_gather(tbase + b + AHEAD, bg)
            wait_gather(b)
            issue_scatters(tbase + b, b)

        for b in range(NBUF):
            wait_scatters(b)

        plsc.subcore_barrier()
        obase = cid * SPAD + rbase
        pltpu.sync_copy(ssum.at[pl.ds(rbase, ROWS_PER_TILE)],
                        psum_hbm.at[pl.ds(obase, ROWS_PER_TILE),
                                    pl.ds(0, OUT_C)])
        pltpu.sync_copy(scnt.at[pl.ds(rbase, ROWS_PER_TILE)],
                        pcnt_hbm.at[pl.ds(obase, ROWS_PER_TILE)])

    return sc_aggregate


def kernel(x, W, u, c, bias, adj):
    del u, c
    h = _matmul(x, W)

    psum, pcnt = _make_sc_aggregate(adj.shape[1])(adj, h)

    return _finalize(psum, pcnt, h, bias.reshape(1, OUT_C))
